# baseline (speedup 1.0000x reference)
import functools

import jax
import jax.numpy as jnp
from jax import lax
from jax.experimental import pallas as pl
from jax.experimental.pallas import tpu as pltpu
from jax.experimental.pallas import tpu_sc as plsc

D = 128
AW = 144
BR = 400

NC = 2
NS = 16
LN = 16
CH = 80
ZCH = 125


def _tc_body(first, h_or_acc_ref, Wt_ref, b_ref, asd_ref, abv_ref,
             zaug_ref, d_ref):
    if first:
        h = h_or_acc_ref[...]
    else:
        acc = h_or_acc_ref[...]
        num = acc[0, :, :D] + acc[1, :, :D]
        den = acc[0, :, D:D + 1] + acc[1, :, D:D + 1]
        h = jnp.maximum(jnp.where(den > 0.0, num / den, 0.0), 0.0)
    zb = jnp.dot(h, Wt_ref[...], preferred_element_type=jnp.float32) + b_ref[...]
    sd = jnp.dot(zb, asd_ref[...], preferred_element_type=jnp.float32) + abv_ref[...]
    lane = lax.broadcasted_iota(jnp.int32, (BR, AW - D), 1)
    tail = jnp.where(lane == 0, 1.0, jnp.where(lane == 1, sd[:, 0:1], 0.0))
    zaug_ref[...] = jnp.concatenate([zb, tail.astype(jnp.float32)], axis=1)
    d_ref[...] = sd[:, 1:2]


def _tc_layer(n, first, h_or_acc, Wt, b2, asd, abv):
    grid = (n // BR,)
    if first:
        spec0 = pl.BlockSpec((BR, D), lambda i: (i, 0))
    else:
        spec0 = pl.BlockSpec((2, BR, AW), lambda i: (0, i, 0))
    out = pl.pallas_call(
        functools.partial(_tc_body, first),
        grid=grid,
        in_specs=[
            spec0,
            pl.BlockSpec((D, D), lambda i: (0, 0)),
            pl.BlockSpec((1, D), lambda i: (0, 0)),
            pl.BlockSpec((D, 2), lambda i: (0, 0)),
            pl.BlockSpec((1, 2), lambda i: (0, 0)),
        ],
        out_specs=[
            pl.BlockSpec((BR, AW), lambda i: (i, 0)),
            pl.BlockSpec((BR, 1), lambda i: (i, 0)),
        ],
        out_shape=[
            jax.ShapeDtypeStruct((n, AW), jnp.float32),
            jax.ShapeDtypeStruct((n, 1), jnp.float32),
        ],
    )(h_or_acc, Wt, b2, asd, abv)
    return out


def _combine_body(acc_ref, out_ref):
    acc = acc_ref[...]
    num = acc[0, :, :D] + acc[1, :, :D]
    den = acc[0, :, D:D + 1] + acc[1, :, D:D + 1]
    out_ref[...] = jnp.where(den > 0.0, num / den, 0.0)


def _combine(n, acc):
    return pl.pallas_call(
        _combine_body,
        grid=(n // BR,),
        in_specs=[pl.BlockSpec((2, BR, AW), lambda i: (0, i, 0))],
        out_specs=pl.BlockSpec((BR, D), lambda i: (i, 0)),
        out_shape=jax.ShapeDtypeStruct((n, D), jnp.float32),
    )(acc)


def _sc_edge_pass(n, zaug, dvec, ei, zfull):
    e_total = ei.shape[1]
    per_w = e_total // (NC * NS)
    n_chunks = per_w // CH

    mesh = plsc.VectorSubcoreMesh(core_axis_name="c", subcore_axis_name="s")

    @functools.partial(
        pl.kernel,
        out_type=jax.ShapeDtypeStruct((NC, n, AW), jnp.float32),
        mesh=mesh,
        scratch_types=[
            pltpu.VMEM((n,), jnp.float32),
            pltpu.VMEM((2, CH), jnp.int32),
            pltpu.VMEM((2, CH), jnp.int32),
            pltpu.VMEM((CH, AW), jnp.float32),
            pltpu.VMEM((CH, AW), jnp.float32),
            pltpu.VMEM_SHARED((n, AW), jnp.float32),
            pltpu.SemaphoreType.DMA,
            pltpu.SemaphoreType.DMA,
            pltpu.SemaphoreType.DMA,
            pltpu.SemaphoreType.DMA,
        ],
        compiler_params=pltpu.CompilerParams(
            use_tc_tiling_on_sc=False, needs_layout_passes=False),
    )
    def sc_kernel(zaug_hbm, d_hbm, ei_hbm, z_hbm, out_hbm,
                  d_t, ei_a, ei_b, rows_a, rows_b, acc_sh, ga, gb, sa, sb):
        cid = lax.axis_index("c")
        sid = lax.axis_index("s")
        wid = sid * NC + cid
        nzc = n // CH
        A = (ei_a, rows_a, ga)
        B = (ei_b, rows_b, gb)

        def prefetch(buf, kc):
            ei_v, rows_v, gsem = buf
            base = wid * per_w + kc * CH
            pltpu.sync_copy(ei_hbm.at[:, pl.ds(base, CH)], ei_v)
            pltpu.async_copy(zaug_hbm.at[ei_v.at[0]], rows_v, gsem)

        def wait_gather(buf):
            ei_v, rows_v, gsem = buf
            pltpu.make_async_copy(zaug_hbm.at[ei_v.at[0]], rows_v, gsem).wait()

        def scale(buf):
            ei_v, rows_v, _ = buf
            col_s = jnp.full((LN,), D + 1, jnp.int32)
            for g in range(CH // LN):
                rr = lax.iota(jnp.int32, LN) + (g * LN)
                sg = plsc.load_gather(rows_v, [rr, col_s])
                dg = plsc.load_gather(d_t, [ei_v[1, pl.ds(g * LN, LN)]])
                tt = sg + dg
                wg = jnp.exp(jnp.maximum(tt, 0.01 * tt))
                for lane in range(LN):
                    ws = wg[lane]
                    e = g * LN + lane
                    for j in range(AW // LN):
                        rows_v[e, pl.ds(j * LN, LN)] = (
                            ws * rows_v[e, pl.ds(j * LN, LN)])

        def issue_scatter(buf, ssem):
            ei_v, rows_v, _ = buf
            return pltpu.async_copy(rows_v, acc_sh.at[ei_v.at[1]], ssem,
                                    add=True)

        pltpu.sync_copy(d_hbm, d_t)

        @pl.loop(0, nzc)
        def _(c):
            @pl.when(lax.rem(c, NS) == sid)
            def _():
                pltpu.sync_copy(z_hbm.at[pl.ds(c * CH, CH)],
                                acc_sh.at[pl.ds(c * CH, CH)])

        prefetch(A, 0)
        plsc.subcore_barrier()

        @pl.loop(0, (n_chunks - 1) // 2)
        def _(k2):
            kc = 2 * k2
            prefetch(B, kc + 1)
            wait_gather(A)
            scale(A)
            d_a = issue_scatter(A, sa)
            wait_gather(B)
            scale(B)
            d_b = issue_scatter(B, sb)
            d_a.wait()
            prefetch(A, kc + 2)
            d_b.wait()

        wait_gather(A)
        scale(A)
        issue_scatter(A, sa).wait()
        plsc.subcore_barrier()

        @pl.loop(0, nzc)
        def _(c):
            @pl.when(lax.rem(c, NS) == sid)
            def _():
                pltpu.sync_copy(acc_sh.at[pl.ds(c * CH, CH)],
                                out_hbm.at[cid, pl.ds(c * CH, CH)])

    return sc_kernel(zaug, dvec, ei, zfull)


def kernel(t, x, edge_index, W1, b1, a1, ab1, W2, b2, a2, ab2, W3, b3, a3, ab3):
    n = x.shape[0]
    ei = edge_index.astype(jnp.int32)
    zfull = jnp.zeros((n, AW), jnp.float32)

    def prep(W, b, a, ab):
        Wt = W.T
        asd = jnp.stack([a[0, :D], a[0, D:]], axis=1)
        abv = jnp.stack([ab, jnp.zeros_like(ab)], axis=1)
        return Wt, b.reshape(1, D), asd, abv

    acc = None
    for i, (W, b, a, ab) in enumerate(
            [(W1, b1, a1, ab1), (W2, b2, a2, ab2), (W3, b3, a3, ab3)]):
        Wt, b2d, asd, abv = prep(W, b, a, ab)
        zaug, d2 = _tc_layer(n, i == 0, x if i == 0 else acc, Wt, b2d, asd, abv)
        acc = _sc_edge_pass(n, zaug, d2.reshape(n), ei, zfull)
    return _combine(n, acc)
